# baseline (speedup 1.0000x reference)
import functools

import jax
import jax.numpy as jnp
from jax import lax
from jax.experimental import pallas as pl
from jax.experimental.pallas import tpu as pltpu
from jax.experimental.pallas import tpu_sc as plsc

NC = 2
NS = 16
NW = NC * NS
LANES = 128
NUM_GRAPHS = 16
NBUF = 2
CH = 64


def _make_deg_kernel(kch, ndeg):
  rps = ndeg // NS
  mesh = plsc.VectorSubcoreMesh(core_axis_name="c", subcore_axis_name="s")

  @functools.partial(
      pl.kernel,
      mesh=mesh,
      out_type=jax.ShapeDtypeStruct((NC * ndeg,), jnp.float32),
      scratch_types=[
          pltpu.VMEM((kch, 2 * CH), jnp.int32),
          pltpu.VMEM((2 * CH,), jnp.float32),
          pltpu.VMEM_SHARED((ndeg,), jnp.float32),
      ],
  )
  def k(dst_hbm, zeros_hbm, out_hbm, dst_v, ones_v, hist_sh):
    cid = lax.axis_index("c")
    sid = lax.axis_index("s")
    wid = sid * NC + cid
    pltpu.sync_copy(dst_hbm.at[wid], dst_v)
    for j in range(2 * CH // 16):
      ones_v[pl.ds(j * 16, 16)] = jnp.ones((16,), jnp.float32)
    pltpu.sync_copy(zeros_hbm.at[pl.ds(sid * rps, rps)],
                    hist_sh.at[pl.ds(sid * rps, rps)])
    plsc.subcore_barrier()

    def body(r, carry):
      pltpu.sync_copy(ones_v, hist_sh.at[dst_v.at[r]], add=True)
      return carry

    lax.fori_loop(0, kch, body, 0)
    plsc.subcore_barrier()
    pltpu.sync_copy(hist_sh.at[pl.ds(sid * rps, rps)],
                    out_hbm.at[pl.ds(cid * ndeg + sid * rps, rps)])

  return k


def _make_agg_kernel(kch, n_pad):
  assert kch >= 2
  rps = n_pad // NS
  mesh = plsc.VectorSubcoreMesh(core_axis_name="c", subcore_axis_name="s")

  @functools.partial(
      pl.kernel,
      mesh=mesh,
      out_type=jax.ShapeDtypeStruct((NC, n_pad, LANES), jnp.float32),
      scratch_types=[
          pltpu.VMEM((kch, 2 * CH), jnp.int32),
          pltpu.VMEM((kch, 2 * CH), jnp.int32),
          pltpu.VMEM((NBUF, CH, LANES), jnp.float32),
          pltpu.VMEM_SHARED((n_pad, LANES), jnp.float32),
      ] + [pltpu.SemaphoreType.DMA] * NBUF,
  )
  def k(table_hbm, src_hbm, dst_hbm, zeros_hbm, out_hbm, src_v, dst_v,
        rows_v, acc_sh, *sems):
    cid = lax.axis_index("c")
    sid = lax.axis_index("s")
    wid = sid * NC + cid
    pltpu.sync_copy(src_hbm.at[wid], src_v)
    pltpu.sync_copy(dst_hbm.at[wid], dst_v)

    @pl.when(cid == 0)
    def _():
      pltpu.sync_copy(table_hbm.at[pl.ds(sid * rps, rps)],
                      acc_sh.at[pl.ds(sid * rps, rps)])

    @pl.when(cid != 0)
    def _():
      pltpu.sync_copy(zeros_hbm.at[pl.ds(sid * rps, rps)],
                      acc_sh.at[pl.ds(sid * rps, rps)])

    plsc.subcore_barrier()

    for b in range(NBUF):
      pltpu.async_copy(table_hbm.at[src_v.at[0, pl.ds(b * CH, CH)]],
                       rows_v.at[b], sems[b])

    def outer(r, carry):
      for b in range(NBUF):
        pltpu.make_async_copy(table_hbm.at[src_v.at[r, pl.ds(b * CH, CH)]],
                              rows_v.at[b], sems[b]).wait()
        pltpu.sync_copy(rows_v.at[b],
                        acc_sh.at[dst_v.at[r, pl.ds(b * CH, CH)]], add=True)
        pltpu.async_copy(table_hbm.at[src_v.at[r + 1, pl.ds(b * CH, CH)]],
                         rows_v.at[b], sems[b])
      return carry

    lax.fori_loop(0, kch - 1, outer, 0)
    for b in range(NBUF):
      pltpu.make_async_copy(table_hbm.at[src_v.at[kch - 1, pl.ds(b * CH, CH)]],
                            rows_v.at[b], sems[b]).wait()
      pltpu.sync_copy(rows_v.at[b],
                      acc_sh.at[dst_v.at[kch - 1, pl.ds(b * CH, CH)]],
                      add=True)
    plsc.subcore_barrier()
    pltpu.sync_copy(acc_sh.at[pl.ds(sid * rps, rps)],
                    out_hbm.at[cid, pl.ds(sid * rps, rps)])

  return k


def _mm(xp, w, bm=1024):
  m, kdim = xp.shape
  h = w.shape[1]

  def body(x_ref, w_ref, o_ref):
    o_ref[...] = jnp.dot(x_ref[...], w_ref[...],
                         preferred_element_type=jnp.float32)

  return pl.pallas_call(
      body,
      grid=(m // bm,),
      in_specs=[
          pl.BlockSpec((bm, kdim), lambda i: (i, 0)),
          pl.BlockSpec((kdim, h), lambda i: (0, 0)),
      ],
      out_specs=pl.BlockSpec((bm, h), lambda i: (i, 0)),
      out_shape=jax.ShapeDtypeStruct((m, h), jnp.float32),
  )(xp, w)


def _scale(xw, degp, bm=1024):
  m, h = xw.shape

  def body(xw_ref, d0_ref, d1_ref, o_ref):
    dinv = lax.rsqrt(d0_ref[0] + d1_ref[0] + 1.0)
    o_ref[...] = xw_ref[...] * dinv

  return pl.pallas_call(
      body,
      grid=(m // bm,),
      in_specs=[
          pl.BlockSpec((bm, h), lambda i: (i, 0)),
          pl.BlockSpec((1, bm, 1), lambda i: (0, i, 0)),
          pl.BlockSpec((1, bm, 1), lambda i: (1, i, 0)),
      ],
      out_specs=pl.BlockSpec((bm, h), lambda i: (i, 0)),
      out_shape=jax.ShapeDtypeStruct((m, h), jnp.float32),
  )(xw, degp, degp)


def _combine_mm(agg, degp, b_row, w2, bm=1024):
  m, h = agg.shape[1:]

  def body(p0_ref, p1_ref, d0_ref, d1_ref, b_ref, w_ref, o_ref):
    dinv = lax.rsqrt(d0_ref[0] + d1_ref[0] + 1.0)
    hcomb = dinv * (p0_ref[0] + p1_ref[0]) + b_ref[...]
    hcomb = jnp.maximum(hcomb, 0.0)
    o_ref[...] = jnp.dot(hcomb, w_ref[...],
                         preferred_element_type=jnp.float32) * dinv

  return pl.pallas_call(
      body,
      grid=(m // bm,),
      in_specs=[
          pl.BlockSpec((1, bm, h), lambda i: (0, i, 0)),
          pl.BlockSpec((1, bm, h), lambda i: (1, i, 0)),
          pl.BlockSpec((1, bm, 1), lambda i: (0, i, 0)),
          pl.BlockSpec((1, bm, 1), lambda i: (1, i, 0)),
          pl.BlockSpec((1, h), lambda i: (0, 0)),
          pl.BlockSpec((h, h), lambda i: (0, 0)),
      ],
      out_specs=pl.BlockSpec((bm, h), lambda i: (i, 0)),
      out_shape=jax.ShapeDtypeStruct((m, h), jnp.float32),
  )(agg, agg, degp, degp, b_row, w2)


def _final(agg, degp, b_row, batch_col, wc, bc_row, bm=1024):
  m, h = agg.shape[1:]
  c = wc.shape[1]
  nb = m // bm

  def body(p0_ref, p1_ref, d0_ref, d1_ref, b_ref, bt_ref, wc_ref, bc_ref,
           o_ref, sums, counts):
    i = pl.program_id(0)

    @pl.when(i == 0)
    def _():
      sums[...] = jnp.zeros_like(sums)
      counts[...] = jnp.zeros_like(counts)

    dinv = lax.rsqrt(d0_ref[0] + d1_ref[0] + 1.0)
    h2 = dinv * (p0_ref[0] + p1_ref[0]) + b_ref[...]
    oh = (bt_ref[...] == lax.broadcasted_iota(jnp.int32, (bm, NUM_GRAPHS), 1)
          ).astype(jnp.float32)
    sums[...] += lax.dot_general(oh, h2, (((0,), (0,)), ((), ())),
                                 preferred_element_type=jnp.float32)
    counts[...] += lax.dot_general(oh, jnp.ones((bm, 1), jnp.float32),
                                   (((0,), (0,)), ((), ())),
                                   preferred_element_type=jnp.float32)

    @pl.when(i == nb - 1)
    def _():
      o_ref[...] = (jnp.dot(sums[...], wc_ref[...],
                            preferred_element_type=jnp.float32)
                    / jnp.maximum(counts[...], 1.0)) + bc_ref[...]

  return pl.pallas_call(
      body,
      grid=(nb,),
      in_specs=[
          pl.BlockSpec((1, bm, h), lambda i: (0, i, 0)),
          pl.BlockSpec((1, bm, h), lambda i: (1, i, 0)),
          pl.BlockSpec((1, bm, 1), lambda i: (0, i, 0)),
          pl.BlockSpec((1, bm, 1), lambda i: (1, i, 0)),
          pl.BlockSpec((1, h), lambda i: (0, 0)),
          pl.BlockSpec((bm, 1), lambda i: (i, 0)),
          pl.BlockSpec((h, c), lambda i: (0, 0)),
          pl.BlockSpec((1, c), lambda i: (0, 0)),
      ],
      out_specs=pl.BlockSpec((NUM_GRAPHS, c), lambda i: (0, 0)),
      out_shape=jax.ShapeDtypeStruct((NUM_GRAPHS, c), jnp.float32),
      scratch_shapes=[
          pltpu.VMEM((NUM_GRAPHS, h), jnp.float32),
          pltpu.VMEM((NUM_GRAPHS, 1), jnp.float32),
      ],
  )(agg, agg, degp, degp, b_row, batch_col, wc, bc_row)


def kernel(x, edge_index, batch, W1, b1, W2, b2, Wc, bc):
  n, d = x.shape
  e = edge_index.shape[1]

  n_pad = -(-n // (NS * LANES)) * (NS * LANES)
  kch = -(-e // (NW * 2 * CH))
  e_pad = NW * kch * 2 * CH

  xp = jnp.pad(x, ((0, n_pad - n), (0, 0)))
  pad_ids = n + (jnp.arange(e_pad - e, dtype=jnp.int32) % (n_pad - n))
  srcp = jnp.concatenate([edge_index[0], pad_ids]).reshape(NW, kch, 2 * CH)
  dstp = jnp.concatenate([edge_index[1], pad_ids]).reshape(NW, kch, 2 * CH)
  zeros = jnp.zeros((n_pad, LANES), jnp.float32)
  zeros_deg = jnp.zeros((n_pad,), jnp.float32)
  batch_col = jnp.pad(batch, (0, n_pad - n),
                      constant_values=NUM_GRAPHS).reshape(n_pad, 1)

  deg_flat = _make_deg_kernel(kch, n_pad)(dstp, zeros_deg)
  degp = deg_flat.reshape(NC, n_pad, 1)

  agg = _make_agg_kernel(kch, n_pad)

  xw = _mm(xp, W1)
  h1p = _scale(xw, degp)
  agg1 = agg(h1p, srcp, dstp, zeros)
  h2p = _combine_mm(agg1, degp, b1.reshape(1, -1), W2)
  agg2 = agg(h2p, srcp, dstp, zeros)
  return _final(agg2, degp, b2.reshape(1, -1), batch_col, Wc, bc.reshape(1, -1))


# speedup vs baseline: 1.2074x; 1.2074x over previous
import functools

import jax
import jax.numpy as jnp
from jax import lax
from jax.experimental import pallas as pl
from jax.experimental.pallas import tpu as pltpu
from jax.experimental.pallas import tpu_sc as plsc

NC = 2
NS = 16
NW = NC * NS
LANES = 128
NUM_GRAPHS = 16
NBUF = 3
CH = 64


def _make_deg_kernel(kch, ndeg):
  rps = ndeg // NS
  mesh = plsc.VectorSubcoreMesh(core_axis_name="c", subcore_axis_name="s")

  @functools.partial(
      pl.kernel,
      mesh=mesh,
      out_type=jax.ShapeDtypeStruct((NC * ndeg,), jnp.float32),
      scratch_types=[
          pltpu.VMEM((kch, 2 * CH), jnp.int32),
          pltpu.VMEM((2 * CH,), jnp.float32),
          pltpu.VMEM_SHARED((ndeg,), jnp.float32),
      ],
  )
  def k(dst_hbm, zeros_hbm, out_hbm, dst_v, ones_v, hist_sh):
    cid = lax.axis_index("c")
    sid = lax.axis_index("s")
    wid = sid * NC + cid
    pltpu.sync_copy(dst_hbm.at[wid], dst_v)
    for j in range(2 * CH // 16):
      ones_v[pl.ds(j * 16, 16)] = jnp.ones((16,), jnp.float32)
    pltpu.sync_copy(zeros_hbm.at[pl.ds(sid * rps, rps)],
                    hist_sh.at[pl.ds(sid * rps, rps)])
    plsc.subcore_barrier()

    def body(r, carry):
      pltpu.sync_copy(ones_v, hist_sh.at[dst_v.at[r]], add=True)
      return carry

    lax.fori_loop(0, kch, body, 0)
    plsc.subcore_barrier()
    pltpu.sync_copy(hist_sh.at[pl.ds(sid * rps, rps)],
                    out_hbm.at[pl.ds(cid * ndeg + sid * rps, rps)])

  return k


def _make_agg_kernel(kch, n_pad):
  assert kch >= 2
  rps = n_pad // NS
  mesh = plsc.VectorSubcoreMesh(core_axis_name="c", subcore_axis_name="s")

  @functools.partial(
      pl.kernel,
      mesh=mesh,
      out_type=jax.ShapeDtypeStruct((NC, n_pad, LANES), jnp.float32),
      scratch_types=[
          pltpu.VMEM((kch, 2 * CH), jnp.int32),
          pltpu.VMEM((kch, 2 * CH), jnp.int32),
          pltpu.VMEM((NBUF, CH, LANES), jnp.float32),
          pltpu.VMEM_SHARED((n_pad, LANES), jnp.float32),
      ] + [pltpu.SemaphoreType.DMA] * NBUF,
  )
  def k(table_hbm, src_hbm, dst_hbm, zeros_hbm, out_hbm, src_v, dst_v,
        rows_v, acc_sh, *sems):
    cid = lax.axis_index("c")
    sid = lax.axis_index("s")
    wid = sid * NC + cid
    pltpu.sync_copy(src_hbm.at[wid], src_v)
    pltpu.sync_copy(dst_hbm.at[wid], dst_v)

    @pl.when(cid == 0)
    def _():
      pltpu.sync_copy(table_hbm.at[pl.ds(sid * rps, rps)],
                      acc_sh.at[pl.ds(sid * rps, rps)])

    @pl.when(cid != 0)
    def _():
      pltpu.sync_copy(zeros_hbm.at[pl.ds(sid * rps, rps)],
                      acc_sh.at[pl.ds(sid * rps, rps)])

    plsc.subcore_barrier()

    kc = 2 * kch

    for b in range(NBUF):
      r, p = b // 2, b % 2
      pltpu.async_copy(table_hbm.at[src_v.at[r, pl.ds(p * CH, CH)]],
                       rows_v.at[b], sems[b])

    def outer(o, carry):
      c0 = o * NBUF
      for b in range(NBUF):
        c = c0 + b
        r = c // 2
        sl = pl.ds((c - 2 * r) * CH, CH)
        pltpu.make_async_copy(table_hbm.at[src_v.at[r, sl]],
                              rows_v.at[b], sems[b]).wait()
        pltpu.sync_copy(rows_v.at[b], acc_sh.at[dst_v.at[r, sl]], add=True)
        cn = c + NBUF
        rn = cn // 2
        sln = pl.ds((cn - 2 * rn) * CH, CH)
        pltpu.async_copy(table_hbm.at[src_v.at[rn, sln]], rows_v.at[b],
                         sems[b])
      return carry

    lax.fori_loop(0, kc // NBUF - 1, outer, 0)
    for b in range(NBUF):
      c = kc - NBUF + b
      r, p = c // 2, c % 2
      sl = pl.ds(p * CH, CH)
      pltpu.make_async_copy(table_hbm.at[src_v.at[r, sl]], rows_v.at[b],
                            sems[b]).wait()
      pltpu.sync_copy(rows_v.at[b], acc_sh.at[dst_v.at[r, sl]], add=True)
    plsc.subcore_barrier()
    pltpu.sync_copy(acc_sh.at[pl.ds(sid * rps, rps)],
                    out_hbm.at[cid, pl.ds(sid * rps, rps)])

  return k


def _mm(xp, w, bm=1024):
  m, kdim = xp.shape
  h = w.shape[1]

  def body(x_ref, w_ref, o_ref):
    o_ref[...] = jnp.dot(x_ref[...], w_ref[...],
                         preferred_element_type=jnp.float32)

  return pl.pallas_call(
      body,
      grid=(m // bm,),
      in_specs=[
          pl.BlockSpec((bm, kdim), lambda i: (i, 0)),
          pl.BlockSpec((kdim, h), lambda i: (0, 0)),
      ],
      out_specs=pl.BlockSpec((bm, h), lambda i: (i, 0)),
      out_shape=jax.ShapeDtypeStruct((m, h), jnp.float32),
  )(xp, w)


def _dinv(deg_parts):
  def body(d_ref, o_ref):
    o_ref[...] = lax.rsqrt(jnp.sum(d_ref[...], axis=0) + 1.0)

  return pl.pallas_call(
      body,
      out_shape=jax.ShapeDtypeStruct(deg_parts.shape[1:], jnp.float32),
  )(deg_parts)


def _scale(xw, dinv_col, bm=1024):
  m, h = xw.shape

  def body(xw_ref, dv_ref, o_ref):
    o_ref[...] = xw_ref[...] * dv_ref[...]

  return pl.pallas_call(
      body,
      grid=(m // bm,),
      in_specs=[
          pl.BlockSpec((bm, h), lambda i: (i, 0)),
          pl.BlockSpec((bm, 1), lambda i: (i, 0)),
      ],
      out_specs=pl.BlockSpec((bm, h), lambda i: (i, 0)),
      out_shape=jax.ShapeDtypeStruct((m, h), jnp.float32),
  )(xw, dinv_col)


def _combine_mm(agg, dinv_col, b_row, w2, bm=1024):
  m, h = agg.shape[1:]

  def body(p0_ref, p1_ref, dv_ref, b_ref, w_ref, o_ref):
    dinv = dv_ref[...]
    hcomb = dinv * (p0_ref[0] + p1_ref[0]) + b_ref[...]
    hcomb = jnp.maximum(hcomb, 0.0)
    o_ref[...] = jnp.dot(hcomb, w_ref[...],
                         preferred_element_type=jnp.float32) * dinv

  return pl.pallas_call(
      body,
      grid=(m // bm,),
      in_specs=[
          pl.BlockSpec((1, bm, h), lambda i: (0, i, 0)),
          pl.BlockSpec((1, bm, h), lambda i: (1, i, 0)),
          pl.BlockSpec((bm, 1), lambda i: (i, 0)),
          pl.BlockSpec((1, h), lambda i: (0, 0)),
          pl.BlockSpec((h, h), lambda i: (0, 0)),
      ],
      out_specs=pl.BlockSpec((bm, h), lambda i: (i, 0)),
      out_shape=jax.ShapeDtypeStruct((m, h), jnp.float32),
  )(agg, agg, dinv_col, b_row, w2)


def _final(agg, dinv_col, b_row, batch_col, wc, bc_row, bm=1024):
  m, h = agg.shape[1:]
  c = wc.shape[1]
  nb = m // bm

  def body(p0_ref, p1_ref, dv_ref, b_ref, bt_ref, wc_ref, bc_ref,
           o_ref, sums, counts):
    i = pl.program_id(0)

    @pl.when(i == 0)
    def _():
      sums[...] = jnp.zeros_like(sums)
      counts[...] = jnp.zeros_like(counts)

    h2 = dv_ref[...] * (p0_ref[0] + p1_ref[0]) + b_ref[...]
    oh = (bt_ref[...] == lax.broadcasted_iota(jnp.int32, (bm, NUM_GRAPHS), 1)
          ).astype(jnp.float32)
    sums[...] += lax.dot_general(oh, h2, (((0,), (0,)), ((), ())),
                                 preferred_element_type=jnp.float32)
    counts[...] += lax.dot_general(oh, jnp.ones((bm, 1), jnp.float32),
                                   (((0,), (0,)), ((), ())),
                                   preferred_element_type=jnp.float32)

    @pl.when(i == nb - 1)
    def _():
      o_ref[...] = (jnp.dot(sums[...], wc_ref[...],
                            preferred_element_type=jnp.float32)
                    / jnp.maximum(counts[...], 1.0)) + bc_ref[...]

  return pl.pallas_call(
      body,
      grid=(nb,),
      in_specs=[
          pl.BlockSpec((1, bm, h), lambda i: (0, i, 0)),
          pl.BlockSpec((1, bm, h), lambda i: (1, i, 0)),
          pl.BlockSpec((bm, 1), lambda i: (i, 0)),
          pl.BlockSpec((1, h), lambda i: (0, 0)),
          pl.BlockSpec((bm, 1), lambda i: (i, 0)),
          pl.BlockSpec((h, c), lambda i: (0, 0)),
          pl.BlockSpec((1, c), lambda i: (0, 0)),
      ],
      out_specs=pl.BlockSpec((NUM_GRAPHS, c), lambda i: (0, 0)),
      out_shape=jax.ShapeDtypeStruct((NUM_GRAPHS, c), jnp.float32),
      scratch_shapes=[
          pltpu.VMEM((NUM_GRAPHS, h), jnp.float32),
          pltpu.VMEM((NUM_GRAPHS, 1), jnp.float32),
      ],
  )(agg, agg, dinv_col, b_row, batch_col, wc, bc_row)


def kernel(x, edge_index, batch, W1, b1, W2, b2, Wc, bc):
  n, d = x.shape
  e = edge_index.shape[1]

  n_pad = -(-n // (NS * LANES)) * (NS * LANES)
  kch = -(-e // (NW * 2 * CH * NBUF)) * NBUF
  e_pad = NW * kch * 2 * CH

  xp = jnp.pad(x, ((0, n_pad - n), (0, 0)))
  pad_ids = n + (jnp.arange(e_pad - e, dtype=jnp.int32) % (n_pad - n))
  srcp = jnp.concatenate([edge_index[0], pad_ids]).reshape(NW, kch, 2 * CH)
  dstp = jnp.concatenate([edge_index[1], pad_ids]).reshape(NW, kch, 2 * CH)
  zeros = jnp.zeros((n_pad, LANES), jnp.float32)
  zeros_deg = jnp.zeros((n_pad,), jnp.float32)
  batch_col = jnp.pad(batch, (0, n_pad - n),
                      constant_values=NUM_GRAPHS).reshape(n_pad, 1)

  deg_flat = _make_deg_kernel(kch, n_pad)(dstp, zeros_deg)
  dinv_col = _dinv(deg_flat.reshape(NC, n_pad // LANES, LANES)).reshape(
      n_pad, 1)

  agg = _make_agg_kernel(kch, n_pad)

  xw = _mm(xp, W1)
  h1p = _scale(xw, dinv_col)
  agg1 = agg(h1p, srcp, dstp, zeros)
  h2p = _combine_mm(agg1, dinv_col, b1.reshape(1, -1), W2)
  agg2 = agg(h2p, srcp, dstp, zeros)
  return _final(agg2, dinv_col, b2.reshape(1, -1), batch_col, Wc,
                bc.reshape(1, -1))
